# baseline (speedup 1.0000x reference)
import jax
import jax.numpy as jnp
from jax.experimental import pallas as pl

B = 8
N = 4096
K = 5
LOCAL = 10
BINS = 80
R = 256
PS, PL = 256, 128
SWEEPS = 4


def _knn_kernel(pts_ref, ptsT_ref, ptsf_ref, idx_ref, nbh_ref):
    xi = [pts_ref[0, :, d].reshape(R, 1) for d in range(3)]
    xj = [ptsT_ref[0, d, :].reshape(1, N) for d in range(3)]
    d2 = None
    for d in range(3):
        df = xi[d] - xj[d]
        sq = df * df
        d2 = sq if d2 is None else d2 + sq
    iota = jax.lax.broadcasted_iota(jnp.int32, (R, N), 1)
    pts_full = ptsf_ref[0]
    idx_cols = []
    nbh_cols = []
    for _ in range(K):
        minv = jnp.min(d2, axis=1, keepdims=True)
        cand = jnp.where(d2 == minv, iota, N)
        imin = jnp.min(cand, axis=1, keepdims=True)
        onehot = iota == imin
        nbh_cols.append(
            jax.lax.dot_general(
                onehot.astype(jnp.float32),
                pts_full,
                (((1,), (0,)), ((), ())),
                precision=jax.lax.Precision.HIGHEST,
                preferred_element_type=jnp.float32,
            )
        )
        idx_cols.append(imin)
        d2 = jnp.where(onehot, jnp.float32(jnp.inf), d2)
    idx_ref[0] = jnp.concatenate(idx_cols, axis=1)
    nbh_ref[0] = jnp.concatenate(nbh_cols, axis=1)


def _jacobi_rotate(a, v, p, q):
    app, aqq, apq = a[p][p], a[q][q], a[p][q]
    tau = (aqq - app) / (2.0 * apq)
    t = jnp.sign(tau) / (jnp.abs(tau) + jnp.sqrt(1.0 + tau * tau))
    t = jnp.where(apq == 0.0, 0.0, t)
    c = 1.0 / jnp.sqrt(1.0 + t * t)
    s = t * c
    for i in range(3):
        bp = c * a[i][p] - s * a[i][q]
        bq = s * a[i][p] + c * a[i][q]
        a[i][p], a[i][q] = bp, bq
    for j in range(3):
        bp = c * a[p][j] - s * a[q][j]
        bq = s * a[p][j] + c * a[q][j]
        a[p][j], a[q][j] = bp, bq
    for i in range(3):
        bp = c * v[i][p] - s * v[i][q]
        bq = s * v[i][p] + c * v[i][q]
        v[i][p], v[i][q] = bp, bq


def _bf(x):
    return x.astype(jnp.bfloat16).astype(jnp.float32)


def _lrf_kernel(nbhT_ref, sid_ref, nrm_ref):
    nb = [nbhT_ref[c] for c in range(3 * K)]
    inv_k = jnp.float32(0.2)
    mu = []
    for d in range(3):
        s = nb[d]
        for k in range(1, K):
            s = s + nb[k * 3 + d]
        mu.append(s * inv_k)
    diff = [[_bf(nb[k * 3 + d] - mu[d]) for d in range(3)] for k in range(K)]
    cov = {}
    for i in range(3):
        for j in range(i, 3):
            s = diff[0][i] * diff[0][j]
            for k in range(1, K):
                s = s + diff[k][i] * diff[k][j]
            cov[(i, j)] = s * inv_k
    a = [[cov[(min(i, j), max(i, j))] for j in range(3)] for i in range(3)]
    one = jnp.ones_like(a[0][0])
    zero = jnp.zeros_like(a[0][0])
    v = [[one if i == j else zero for j in range(3)] for i in range(3)]
    for _ in range(SWEEPS):
        _jacobi_rotate(a, v, 0, 2)
        _jacobi_rotate(a, v, 2, 1)
        _jacobi_rotate(a, v, 0, 1)
    e = [a[0][0], a[1][1], a[2][2]]
    cols = [[v[i][j] for i in range(3)] for j in range(3)]

    def cswap(j0, j1):
        swap = e[j1] < e[j0]
        e0 = jnp.where(swap, e[j1], e[j0])
        e1 = jnp.where(swap, e[j0], e[j1])
        e[j0], e[j1] = e0, e1
        for i in range(3):
            x0, x1 = cols[j0][i], cols[j1][i]
            cols[j0][i] = jnp.where(swap, x1, x0)
            cols[j1][i] = jnp.where(swap, x0, x1)

    cswap(0, 1)
    cswap(1, 2)
    cswap(0, 1)

    nb_bf = [_bf(x) for x in nb]
    cols_bf = [[_bf(cols[j][i]) for i in range(3)] for j in range(3)]
    for k in range(K):
        bits = []
        for d in range(3):
            p = nb_bf[k * 3 + 0] * cols_bf[d][0]
            p = p + nb_bf[k * 3 + 1] * cols_bf[d][1]
            p = p + nb_bf[k * 3 + 2] * cols_bf[d][2]
            bits.append((p >= 0.0).astype(jnp.int32))
        sid_ref[k] = bits[0] * 4 + bits[1] * 2 + bits[2]
    for i in range(3):
        nrm_ref[i] = cols[0][i]


def _hist_kernel(nrm_ref, nrmT_ref, idx_ref, sid_ref, out_ref):
    ni = [nrm_ref[0, :, d].reshape(R, 1) for d in range(3)]
    nj = [nrmT_ref[0, d, :].reshape(1, N) for d in range(3)]
    gram = ni[0] * nj[0]
    gram = gram + ni[1] * nj[1]
    gram = gram + ni[2] * nj[2]
    iota = jax.lax.broadcasted_iota(jnp.int32, (R, N), 1)
    iota80 = jax.lax.broadcasted_iota(jnp.int32, (R, BINS), 1)
    acc = jnp.zeros((R, BINS), jnp.float32)
    for k in range(K):
        idxk = idx_ref[0, :, k].reshape(R, 1)
        cosk = jnp.sum(
            jnp.where(iota == idxk, gram, 0.0), axis=1, keepdims=True
        )
        nid = jnp.clip(jnp.floor(LOCAL * (cosk + 1.0) / 2.0), 0.0, LOCAL - 1.0)
        sidk = sid_ref[0, :, k].reshape(R, 1)
        binid = (sidk.astype(jnp.float32) * LOCAL + nid).astype(jnp.int32)
        acc = acc + (iota80 == binid).astype(jnp.float32)
    out_ref[0] = acc


def kernel(points, batch):
    pts = points.reshape(B, N, 3)
    ptsT = pts.transpose(0, 2, 1)
    idx, nbh = pl.pallas_call(
        _knn_kernel,
        grid=(B, N // R),
        in_specs=[
            pl.BlockSpec((1, R, 3), lambda b, i: (b, i, 0)),
            pl.BlockSpec((1, 3, N), lambda b, i: (b, 0, 0)),
            pl.BlockSpec((1, N, 3), lambda b, i: (b, 0, 0)),
        ],
        out_specs=[
            pl.BlockSpec((1, R, K), lambda b, i: (b, i, 0)),
            pl.BlockSpec((1, R, 3 * K), lambda b, i: (b, i, 0)),
        ],
        out_shape=[
            jax.ShapeDtypeStruct((B, N, K), jnp.int32),
            jax.ShapeDtypeStruct((B, N, 3 * K), jnp.float32),
        ],
    )(pts, ptsT, pts)

    nbhT = nbh.reshape(B * N, 3 * K).T.reshape(3 * K, PS, PL)
    sidT, nrmT_flat = pl.pallas_call(
        _lrf_kernel,
        out_shape=[
            jax.ShapeDtypeStruct((K, PS, PL), jnp.int32),
            jax.ShapeDtypeStruct((3, PS, PL), jnp.float32),
        ],
    )(nbhT)

    sid = sidT.reshape(K, B, N).transpose(1, 2, 0)
    nrm = nrmT_flat.reshape(3, B, N).transpose(1, 2, 0)
    nrmT = nrmT_flat.reshape(3, B, N).transpose(1, 0, 2)
    out = pl.pallas_call(
        _hist_kernel,
        grid=(B, N // R),
        in_specs=[
            pl.BlockSpec((1, R, 3), lambda b, i: (b, i, 0)),
            pl.BlockSpec((1, 3, N), lambda b, i: (b, 0, 0)),
            pl.BlockSpec((1, R, K), lambda b, i: (b, i, 0)),
            pl.BlockSpec((1, R, K), lambda b, i: (b, i, 0)),
        ],
        out_specs=pl.BlockSpec((1, R, BINS), lambda b, i: (b, i, 0)),
        out_shape=jax.ShapeDtypeStruct((B, N, BINS), jnp.float32),
    )(nrm, nrmT, idx, sid)
    return out.reshape(B * N, BINS)


# speedup vs baseline: 1.8860x; 1.8860x over previous
import functools

import jax
import jax.numpy as jnp
from jax import lax
from jax.experimental import pallas as pl
from jax.experimental.pallas import tpu as pltpu
from jax.experimental.pallas import tpu_sc as plsc

B = 8
N = 4096
K = 5
LOCAL = 10
BINS = 80
R = 256
PS, PL = 256, 128
SWEEPS = 4


def _knn_kernel(pts_ref, ptsT_ref, idx_ref, nbh_ref):
    xi = [pts_ref[0, :, d].reshape(R, 1) for d in range(3)]
    xj = [ptsT_ref[0, d, :].reshape(1, N) for d in range(3)]
    d2 = None
    for d in range(3):
        df = xi[d] - xj[d]
        sq = df * df
        d2 = sq if d2 is None else d2 + sq
    iota = jax.lax.broadcasted_iota(jnp.int32, (R, N), 1)
    idx_cols = []
    nbh_cols = []
    for _ in range(K):
        minv = jnp.min(d2, axis=1, keepdims=True)
        cand = jnp.where(d2 == minv, iota, N)
        imin = jnp.min(cand, axis=1, keepdims=True)
        onehot = iota == imin
        for d in range(3):
            nbh_cols.append(
                jnp.sum(jnp.where(onehot, xj[d], 0.0), axis=1, keepdims=True)
            )
        idx_cols.append(imin)
        d2 = jnp.where(onehot, jnp.float32(jnp.inf), d2)
    idx_ref[0] = jnp.concatenate(idx_cols, axis=1)
    nbh_ref[0] = jnp.concatenate(nbh_cols, axis=1)


def _jacobi_rotate(a, v, p, q):
    app, aqq, apq = a[p][p], a[q][q], a[p][q]
    tau = (aqq - app) / (2.0 * apq)
    t = jnp.sign(tau) / (jnp.abs(tau) + jnp.sqrt(1.0 + tau * tau))
    t = jnp.where(apq == 0.0, 0.0, t)
    c = 1.0 / jnp.sqrt(1.0 + t * t)
    s = t * c
    for i in range(3):
        bp = c * a[i][p] - s * a[i][q]
        bq = s * a[i][p] + c * a[i][q]
        a[i][p], a[i][q] = bp, bq
    for j in range(3):
        bp = c * a[p][j] - s * a[q][j]
        bq = s * a[p][j] + c * a[q][j]
        a[p][j], a[q][j] = bp, bq
    for i in range(3):
        bp = c * v[i][p] - s * v[i][q]
        bq = s * v[i][p] + c * v[i][q]
        v[i][p], v[i][q] = bp, bq


def _bf(x):
    return x.astype(jnp.bfloat16).astype(jnp.float32)


def _lrf_kernel(nbhT_ref, sid_ref, nrm_ref):
    nb = [nbhT_ref[c] for c in range(3 * K)]
    inv_k = jnp.float32(0.2)
    mu = []
    for d in range(3):
        s = nb[d]
        for k in range(1, K):
            s = s + nb[k * 3 + d]
        mu.append(s * inv_k)
    diff = [[_bf(nb[k * 3 + d] - mu[d]) for d in range(3)] for k in range(K)]
    cov = {}
    for i in range(3):
        for j in range(i, 3):
            s = diff[0][i] * diff[0][j]
            for k in range(1, K):
                s = s + diff[k][i] * diff[k][j]
            cov[(i, j)] = s * inv_k
    a = [[cov[(min(i, j), max(i, j))] for j in range(3)] for i in range(3)]
    one = jnp.ones_like(a[0][0])
    zero = jnp.zeros_like(a[0][0])
    v = [[one if i == j else zero for j in range(3)] for i in range(3)]
    for _ in range(SWEEPS):
        _jacobi_rotate(a, v, 0, 2)
        _jacobi_rotate(a, v, 2, 1)
        _jacobi_rotate(a, v, 0, 1)
    e = [a[0][0], a[1][1], a[2][2]]
    cols = [[v[i][j] for i in range(3)] for j in range(3)]

    def cswap(j0, j1):
        swap = e[j1] < e[j0]
        e0 = jnp.where(swap, e[j1], e[j0])
        e1 = jnp.where(swap, e[j0], e[j1])
        e[j0], e[j1] = e0, e1
        for i in range(3):
            x0, x1 = cols[j0][i], cols[j1][i]
            cols[j0][i] = jnp.where(swap, x1, x0)
            cols[j1][i] = jnp.where(swap, x0, x1)

    cswap(0, 1)
    cswap(1, 2)
    cswap(0, 1)

    nb_bf = [_bf(x) for x in nb]
    cols_bf = [[_bf(cols[j][i]) for i in range(3)] for j in range(3)]
    for k in range(K):
        bits = []
        for d in range(3):
            p = nb_bf[k * 3 + 0] * cols_bf[d][0]
            p = p + nb_bf[k * 3 + 1] * cols_bf[d][1]
            p = p + nb_bf[k * 3 + 2] * cols_bf[d][2]
            bits.append((p >= 0.0).astype(jnp.int32))
        sid_ref[k] = bits[0] * 4 + bits[1] * 2 + bits[2]
    for i in range(3):
        nrm_ref[i] = cols[0][i]


def _sc_gather_normals(tab, idx_flat):
    info = plsc.get_sparse_core_info()
    nw = info.num_cores * info.num_subcores
    btot = idx_flat.shape[0]
    b_per_w = btot // nw
    per_cloud = N * K
    npts = B * N
    mesh = plsc.VectorSubcoreMesh(core_axis_name="c", subcore_axis_name="s")

    @functools.partial(
        pl.kernel,
        mesh=mesh,
        compiler_params=pltpu.CompilerParams(needs_layout_passes=False),
        out_type=jax.ShapeDtypeStruct((3 * btot,), jnp.float32),
        scratch_types=[
            pltpu.VMEM((b_per_w,), jnp.int32),
            pltpu.VMEM((N,), jnp.float32),
            pltpu.VMEM((N,), jnp.float32),
            pltpu.VMEM((N,), jnp.float32),
            pltpu.VMEM((b_per_w,), jnp.float32),
            pltpu.VMEM((b_per_w,), jnp.float32),
            pltpu.VMEM((b_per_w,), jnp.float32),
        ],
    )
    def k(tab_hbm, idx_hbm, out_hbm, idx_v, t0, t1, t2, o0, o1, o2):
        wid = lax.axis_index("s") * info.num_cores + lax.axis_index("c")
        base = wid * b_per_w
        cloud = base // per_cloud
        pltpu.sync_copy(idx_hbm.at[pl.ds(base, b_per_w)], idx_v)
        tv = (t0, t1, t2)
        ov = (o0, o1, o2)
        for d in range(3):
            pltpu.sync_copy(tab_hbm.at[pl.ds(d * npts + cloud * N, N)], tv[d])

        @pl.loop(0, b_per_w // 16)
        def body(i):
            idx_vec = idx_v[pl.ds(i * 16, 16)]
            for d in range(3):
                ov[d][pl.ds(i * 16, 16)] = plsc.load_gather(tv[d], [idx_vec])
        for d in range(3):
            pltpu.sync_copy(ov[d], out_hbm.at[pl.ds(d * btot + base, b_per_w)])

    return k(tab, idx_flat)


def _hist_kernel(nrm_ref, nbn_ref, sid_ref, out_ref):
    ni = [nrm_ref[0, :, d].reshape(R, 1) for d in range(3)]
    iota80 = jax.lax.broadcasted_iota(jnp.int32, (R, BINS), 1)
    acc = jnp.zeros((R, BINS), jnp.float32)
    for k in range(K):
        bn = [nbn_ref[0, :, k * 3 + d].reshape(R, 1) for d in range(3)]
        cosk = ni[0] * bn[0]
        cosk = cosk + ni[1] * bn[1]
        cosk = cosk + ni[2] * bn[2]
        nid = jnp.clip(jnp.floor(LOCAL * (cosk + 1.0) / 2.0), 0.0, LOCAL - 1.0)
        sidk = sid_ref[0, :, k].reshape(R, 1)
        binid = (sidk.astype(jnp.float32) * LOCAL + nid).astype(jnp.int32)
        acc = acc + (iota80 == binid).astype(jnp.float32)
    out_ref[0] = acc


def kernel(points, batch):
    pts = points.reshape(B, N, 3)
    ptsT = pts.transpose(0, 2, 1)
    idx, nbh = pl.pallas_call(
        _knn_kernel,
        grid=(B, N // R),
        in_specs=[
            pl.BlockSpec((1, R, 3), lambda b, i: (b, i, 0)),
            pl.BlockSpec((1, 3, N), lambda b, i: (b, 0, 0)),
        ],
        out_specs=[
            pl.BlockSpec((1, R, K), lambda b, i: (b, i, 0)),
            pl.BlockSpec((1, R, 3 * K), lambda b, i: (b, i, 0)),
        ],
        out_shape=[
            jax.ShapeDtypeStruct((B, N, K), jnp.int32),
            jax.ShapeDtypeStruct((B, N, 3 * K), jnp.float32),
        ],
    )(pts, ptsT)

    nbhT = nbh.reshape(B * N, 3 * K).T.reshape(3 * K, PS, PL)
    sidT, nrmT_flat = pl.pallas_call(
        _lrf_kernel,
        out_shape=[
            jax.ShapeDtypeStruct((K, PS, PL), jnp.int32),
            jax.ShapeDtypeStruct((3, PS, PL), jnp.float32),
        ],
    )(nbhT)

    sid = sidT.reshape(K, B, N).transpose(1, 2, 0)
    nrm = nrmT_flat.reshape(3, B, N).transpose(1, 2, 0)

    nbnT = _sc_gather_normals(
        nrmT_flat.reshape(3 * B * N), idx.reshape(B * N * K)
    )
    nbn = nbnT.reshape(3, B, N, K).transpose(1, 2, 3, 0).reshape(B, N, 3 * K)

    out = pl.pallas_call(
        _hist_kernel,
        grid=(B, N // R),
        in_specs=[
            pl.BlockSpec((1, R, 3), lambda b, i: (b, i, 0)),
            pl.BlockSpec((1, R, 3 * K), lambda b, i: (b, i, 0)),
            pl.BlockSpec((1, R, K), lambda b, i: (b, i, 0)),
        ],
        out_specs=pl.BlockSpec((1, R, BINS), lambda b, i: (b, i, 0)),
        out_shape=jax.ShapeDtypeStruct((B, N, BINS), jnp.float32),
    )(nrm, nbn, sid)
    return out.reshape(B * N, BINS)
